# baseline (speedup 1.0000x reference)
import functools

import jax
import jax.numpy as jnp
from jax import lax
from jax.experimental import pallas as pl
from jax.experimental.pallas import tpu as pltpu
from jax.experimental.pallas import tpu_sc as plsc

N_USERS_K = 25000
N_NODES_K = 50000
E_K = 800000
EMB_K = 64
TEXT_K = 384
CHUNK = 32
NPLANES = 4

NC = 2
NS = 16
LANES = 16

NNP = 50048
FLAT = NPLANES * NNP
ROWS_PT = NNP // NS

MB = 128
E_PAD = 819200
EPT = E_PAD // NS
BLK_E = 1280
OUTER = EPT // BLK_E
SLAB_E = 320
SLABS = BLK_E // SLAB_E


def _planes_body(text_ref, id_ref, w_ref, b_ref, out_ref):
    t = jnp.dot(text_ref[...], w_ref[...], preferred_element_type=jnp.float32)
    t = t + b_ref[...]
    eid = id_ref[...]
    out_ref[0] = eid[:, :CHUNK]
    out_ref[1] = eid[:, CHUNK:]
    out_ref[2] = t[:, :CHUNK]
    out_ref[3] = t[:, CHUNK:]


def _build_planes(text_feats, emb_id, w, b):
    r = 2000
    grid = N_NODES_K // r
    return pl.pallas_call(
        _planes_body,
        grid=(grid,),
        in_specs=[
            pl.BlockSpec((r, TEXT_K), lambda i: (i, 0)),
            pl.BlockSpec((r, EMB_K), lambda i: (i, 0)),
            pl.BlockSpec((TEXT_K, EMB_K), lambda i: (0, 0)),
            pl.BlockSpec((1, EMB_K), lambda i: (0, 0)),
        ],
        out_specs=pl.BlockSpec((NPLANES, r, CHUNK), lambda i: (0, i, 0)),
        out_shape=jax.ShapeDtypeStruct((NPLANES, NNP, CHUNK), jnp.float32),
    )(text_feats, emb_id, w, b.reshape(1, EMB_K))


def _spmm_body(h0_ref, src_ref, dst_ref, w_ref, zeros_ref, h1_ref, h2_ref,
               acc, srcb, dstb, wb, rows0, rows1, gsem0, gsem1, ssem0, ssem1):
    c = lax.axis_index("c")
    s = lax.axis_index("s")
    rows_bufs = (rows0, rows1)
    gsems = (gsem0, gsem1)
    ssems = (ssem0, ssem1)

    def one_pass(in_ref, out_ref, rowbase):
        pltpu.sync_copy(zeros_ref.at[pl.ds(s * ROWS_PT, ROWS_PT)],
                        acc.at[pl.ds(s * ROWS_PT, ROWS_PT)])
        plsc.subcore_barrier()

        def outer_blk(o, _):
            e0 = s * EPT + o * BLK_E
            pltpu.sync_copy(src_ref.at[pl.ds(e0, BLK_E)], srcb)
            pltpu.sync_copy(dst_ref.at[pl.ds(e0, BLK_E)], dstb)
            pltpu.sync_copy(w_ref.at[pl.ds(e0, BLK_E)], wb)

            def add_base(i, _):
                sl = pl.ds(i * LANES, LANES)
                srcb[sl] = srcb[sl] + rowbase
                return 0
            lax.fori_loop(0, BLK_E // LANES, add_base, 0)

            pltpu.async_copy(in_ref.at[srcb.at[pl.ds(0, SLAB_E)]],
                             rows_bufs[0], gsems[0])

            def micro2(j2, _):
                for b in range(2):
                    j = j2 * 2 + b
                    rows = rows_bufs[b]
                    pltpu.make_async_copy(
                        in_ref.at[srcb.at[pl.ds(SLAB_E * j, SLAB_E)]],
                        rows, gsems[b]).wait()

                    @pl.when(j > 0)
                    def _():
                        pltpu.make_async_copy(
                            rows_bufs[1 - b],
                            acc.at[dstb.at[pl.ds(0, SLAB_E)]],
                            ssems[1 - b]).wait()

                    @pl.when(j + 1 < SLABS)
                    def _():
                        pltpu.async_copy(
                            in_ref.at[srcb.at[pl.ds(SLAB_E * (j + 1), SLAB_E)]],
                            rows_bufs[1 - b], gsems[1 - b])

                    def mul_body(e16, _, j=j):
                        w16 = wb[pl.ds(SLAB_E * j + e16 * LANES, LANES)]
                        for u in range(LANES):
                            e = e16 * LANES + u
                            wv = w16[u]
                            lo = pl.ds(0, LANES)
                            hi = pl.ds(LANES, LANES)
                            rows[e, lo] = rows[e, lo] * wv
                            rows[e, hi] = rows[e, hi] * wv
                        return 0
                    lax.fori_loop(0, SLAB_E // LANES, mul_body, 0)

                    pltpu.async_copy(
                        rows, acc.at[dstb.at[pl.ds(SLAB_E * j, SLAB_E)]],
                        ssems[b], add=True)
                return 0
            lax.fori_loop(0, SLABS // 2, micro2, 0)
            pltpu.make_async_copy(
                rows_bufs[1], acc.at[dstb.at[pl.ds(0, SLAB_E)]],
                ssems[1]).wait()
            return 0
        lax.fori_loop(0, OUTER, outer_blk, 0)
        plsc.subcore_barrier()

        pltpu.sync_copy(acc.at[pl.ds(s * ROWS_PT, ROWS_PT)],
                        out_ref.at[pl.ds(rowbase + s * ROWS_PT, ROWS_PT)])
        plsc.subcore_barrier()

    def chunk_body(k, _):
        rowbase = (2 * c + k) * NNP
        one_pass(h0_ref, h1_ref, rowbase)
        one_pass(h1_ref, h2_ref, rowbase)
        return 0
    lax.fori_loop(0, 2, chunk_body, 0)


def _run_spmm(h0_flat, src2d, dst2d, w2d, zeros):
    mesh = plsc.VectorSubcoreMesh(core_axis_name="c", subcore_axis_name="s")
    kfn = pl.kernel(
        _spmm_body,
        out_type=(
            jax.ShapeDtypeStruct((FLAT, CHUNK), jnp.float32),
            jax.ShapeDtypeStruct((FLAT, CHUNK), jnp.float32),
        ),
        mesh=mesh,
        scratch_types=[
            pltpu.VMEM_SHARED((NNP, CHUNK), jnp.float32),
            pltpu.VMEM((BLK_E,), jnp.int32),
            pltpu.VMEM((BLK_E,), jnp.int32),
            pltpu.VMEM((BLK_E,), jnp.float32),
            pltpu.VMEM((SLAB_E, CHUNK), jnp.float32),
            pltpu.VMEM((SLAB_E, CHUNK), jnp.float32),
            pltpu.SemaphoreType.DMA,
            pltpu.SemaphoreType.DMA,
            pltpu.SemaphoreType.DMA,
            pltpu.SemaphoreType.DMA,
        ],
        compiler_params=pltpu.CompilerParams(use_tc_tiling_on_sc=False),
    )
    return kfn(h0_flat, src2d, dst2d, w2d, zeros)


def _fuse_body(h0_ref, h1_ref, h2_ref, mask_ref, fw_ref, fb_ref, ta_ref,
               out_ref):
    m = (h0_ref[...] + h1_ref[...] + h2_ref[...]) * (1.0 / 3.0)
    id_f = jnp.concatenate([m[0], m[1]], axis=1)
    text_f = jnp.concatenate([m[2], m[3]], axis=1)
    sig_a = jax.nn.sigmoid(ta_ref[0, 0])
    text_f = text_f * (1.0 + mask_ref[...] * sig_a)
    cat = jnp.concatenate([id_f, text_f], axis=1)
    gate = jax.nn.sigmoid(
        jnp.dot(cat, fw_ref[...], preferred_element_type=jnp.float32)
        + fb_ref[...])
    out_ref[...] = gate * id_f + (1.0 - gate) * text_f


def _fuse(h0, h1, h2, maskf, fw, fb, ta):
    r = 2000
    grid = N_NODES_K // r
    return pl.pallas_call(
        _fuse_body,
        grid=(grid,),
        in_specs=[
            pl.BlockSpec((NPLANES, r, CHUNK), lambda i: (0, i, 0)),
            pl.BlockSpec((NPLANES, r, CHUNK), lambda i: (0, i, 0)),
            pl.BlockSpec((NPLANES, r, CHUNK), lambda i: (0, i, 0)),
            pl.BlockSpec((r, 1), lambda i: (i, 0)),
            pl.BlockSpec((2 * EMB_K, EMB_K), lambda i: (0, 0)),
            pl.BlockSpec((1, EMB_K), lambda i: (0, 0)),
            pl.BlockSpec((1, 1), lambda i: (0, 0)),
        ],
        out_specs=pl.BlockSpec((r, EMB_K), lambda i: (i, 0)),
        out_shape=jax.ShapeDtypeStruct((N_NODES_K, EMB_K), jnp.float32),
    )(h0, h1, h2, maskf, fw, fb.reshape(1, EMB_K), ta.reshape(1, 1))


def kernel(edge_index, edge_weight, text_feats, tail_mask, user_emb, item_emb,
           text_proj_W, text_proj_b, fusion_W, fusion_b, tail_amp):
    emb_id = jnp.concatenate([user_emb, item_emb], axis=0)
    h0 = _build_planes(text_feats, emb_id, text_proj_W, text_proj_b)

    pad = E_PAD - E_K
    src1d = jnp.pad(edge_index[1], (0, pad))
    dst1d = jnp.pad(edge_index[0], (0, pad))
    w1d = jnp.pad(edge_weight, (0, pad))
    zeros = jnp.zeros((NNP, CHUNK), jnp.float32)

    h1f, h2f = _run_spmm(h0.reshape(FLAT, CHUNK), src1d, dst1d, w1d, zeros)
    h1 = h1f.reshape(NPLANES, NNP, CHUNK)
    h2 = h2f.reshape(NPLANES, NNP, CHUNK)

    maskf = tail_mask.astype(jnp.float32).reshape(N_NODES_K, 1)
    return _fuse(h0, h1, h2, maskf, fusion_W, fusion_b, tail_amp)


# speedup vs baseline: 1.1359x; 1.1359x over previous
import functools

import jax
import jax.numpy as jnp
from jax import lax
from jax.experimental import pallas as pl
from jax.experimental.pallas import tpu as pltpu
from jax.experimental.pallas import tpu_sc as plsc

N_USERS_K = 25000
N_NODES_K = 50000
E_K = 800000
EMB_K = 64
TEXT_K = 384
CHUNK = 32
NPLANES = 4

NC = 2
NS = 16
LANES = 16

NNP = 50048
FLAT = NPLANES * NNP
ROWS_PT = NNP // NS

MB = 128
E_PAD = 819200
EPT = E_PAD // NS
BLK_E = 1024
OUTER = EPT // BLK_E
SLAB_E = 256
SLABS = BLK_E // SLAB_E


def _planes_body(text_ref, id_ref, w_ref, b_ref, out_ref):
    t = jnp.dot(text_ref[...], w_ref[...], preferred_element_type=jnp.float32)
    t = t + b_ref[...]
    eid = id_ref[...]
    out_ref[0] = eid[:, :CHUNK]
    out_ref[1] = eid[:, CHUNK:]
    out_ref[2] = t[:, :CHUNK]
    out_ref[3] = t[:, CHUNK:]


def _build_planes(text_feats, emb_id, w, b):
    r = 2000
    grid = N_NODES_K // r
    return pl.pallas_call(
        _planes_body,
        grid=(grid,),
        in_specs=[
            pl.BlockSpec((r, TEXT_K), lambda i: (i, 0)),
            pl.BlockSpec((r, EMB_K), lambda i: (i, 0)),
            pl.BlockSpec((TEXT_K, EMB_K), lambda i: (0, 0)),
            pl.BlockSpec((1, EMB_K), lambda i: (0, 0)),
        ],
        out_specs=pl.BlockSpec((NPLANES, r, CHUNK), lambda i: (0, i, 0)),
        out_shape=jax.ShapeDtypeStruct((NPLANES, NNP, CHUNK), jnp.float32),
    )(text_feats, emb_id, w, b.reshape(1, EMB_K))


def _spmm_body(h0_ref, src_ref, dst_ref, w_ref, zeros_ref, h1_ref, h2_ref,
               acc, srcb0, srcb1, dstb0, dstb1, wb0, wb1, rows0, rows1,
               gsem0, gsem1, ssem0, ssem1, isem0, isem1):
    c = lax.axis_index("c")
    s = lax.axis_index("s")
    rows_bufs = (rows0, rows1)
    srcbs = (srcb0, srcb1)
    dstbs = (dstb0, dstb1)
    wbs = (wb0, wb1)
    gsems = (gsem0, gsem1)
    ssems = (ssem0, ssem1)
    isems = (isem0, isem1)

    def load_idx(o, p, rowbase, sync):
        e0 = s * EPT + o * BLK_E
        if sync:
            pltpu.sync_copy(src_ref.at[pl.ds(e0, BLK_E)], srcbs[p])
            pltpu.sync_copy(dst_ref.at[pl.ds(e0, BLK_E)], dstbs[p])
            pltpu.sync_copy(w_ref.at[pl.ds(e0, BLK_E)], wbs[p])
        else:
            pltpu.async_copy(src_ref.at[pl.ds(e0, BLK_E)], srcbs[p], isems[p])
            pltpu.async_copy(dst_ref.at[pl.ds(e0, BLK_E)], dstbs[p], isems[p])
            pltpu.async_copy(w_ref.at[pl.ds(e0, BLK_E)], wbs[p], isems[p])

    def wait_idx(o, p):
        e0 = s * EPT + o * BLK_E
        pltpu.make_async_copy(src_ref.at[pl.ds(e0, BLK_E)], srcbs[p],
                              isems[p]).wait()
        pltpu.make_async_copy(dst_ref.at[pl.ds(e0, BLK_E)], dstbs[p],
                              isems[p]).wait()
        pltpu.make_async_copy(w_ref.at[pl.ds(e0, BLK_E)], wbs[p],
                              isems[p]).wait()

    def add_base(p, rowbase):
        def body(i, _):
            sl = pl.ds(i * LANES, LANES)
            srcbs[p][sl] = srcbs[p][sl] + rowbase
            return 0
        lax.fori_loop(0, BLK_E // LANES, body, 0)

    def one_pass(in_ref, out_ref, rowbase):
        pltpu.sync_copy(zeros_ref.at[pl.ds(s * ROWS_PT, ROWS_PT)],
                        acc.at[pl.ds(s * ROWS_PT, ROWS_PT)])
        plsc.subcore_barrier()

        load_idx(0, 0, rowbase, sync=True)
        add_base(0, rowbase)
        pltpu.async_copy(in_ref.at[srcbs[0].at[pl.ds(0, SLAB_E)]],
                         rows_bufs[0], gsems[0])

        def outer_blk(o2, _):
            for p in range(2):
                o = o2 * 2 + p
                srcb = srcbs[p]
                dstb = dstbs[p]
                wb = wbs[p]

                @pl.when(o + 1 < OUTER)
                def _():
                    load_idx(o + 1, 1 - p, rowbase, sync=False)

                def micro2(j2, _, o=o, p=p, srcb=srcb, dstb=dstb, wb=wb):
                    for b in range(2):
                        j = j2 * 2 + b
                        rows = rows_bufs[b]
                        pltpu.make_async_copy(
                            in_ref.at[srcb.at[pl.ds(SLAB_E * j, SLAB_E)]],
                            rows, gsems[b]).wait()

                        @pl.when((o > 0) | (j > 0))
                        def _():
                            pltpu.make_async_copy(
                                rows_bufs[1 - b],
                                acc.at[dstb.at[pl.ds(0, SLAB_E)]],
                                ssems[1 - b]).wait()

                        @pl.when(j + 1 < SLABS)
                        def _():
                            pltpu.async_copy(
                                in_ref.at[srcb.at[pl.ds(SLAB_E * (j + 1),
                                                        SLAB_E)]],
                                rows_bufs[1 - b], gsems[1 - b])

                        @pl.when(j + 1 == SLABS)
                        def _():
                            @pl.when(o + 1 < OUTER)
                            def _():
                                wait_idx(o + 1, 1 - p)
                                add_base(1 - p, rowbase)
                                pltpu.async_copy(
                                    in_ref.at[srcbs[1 - p].at[
                                        pl.ds(0, SLAB_E)]],
                                    rows_bufs[1 - b], gsems[1 - b])

                        def mul_body(e16, _, j=j, wb=wb):
                            w16 = wb[pl.ds(SLAB_E * j + e16 * LANES, LANES)]
                            for u in range(LANES):
                                e = e16 * LANES + u
                                wv = w16[u]
                                lo = pl.ds(0, LANES)
                                hi = pl.ds(LANES, LANES)
                                rows[e, lo] = rows[e, lo] * wv
                                rows[e, hi] = rows[e, hi] * wv
                            return 0
                        lax.fori_loop(0, SLAB_E // LANES, mul_body, 0)

                        pltpu.async_copy(
                            rows, acc.at[dstb.at[pl.ds(SLAB_E * j, SLAB_E)]],
                            ssems[b], add=True)
                    return 0
                lax.fori_loop(0, SLABS // 2, micro2, 0)
            return 0
        lax.fori_loop(0, OUTER // 2, outer_blk, 0)
        pltpu.make_async_copy(
            rows_bufs[1], acc.at[dstbs[1].at[pl.ds(0, SLAB_E)]],
            ssems[1]).wait()
        plsc.subcore_barrier()

        pltpu.sync_copy(acc.at[pl.ds(s * ROWS_PT, ROWS_PT)],
                        out_ref.at[pl.ds(rowbase + s * ROWS_PT, ROWS_PT)])
        plsc.subcore_barrier()

    def chunk_body(k, _):
        rowbase = (2 * c + k) * NNP
        one_pass(h0_ref, h1_ref, rowbase)
        one_pass(h1_ref, h2_ref, rowbase)
        return 0
    lax.fori_loop(0, 2, chunk_body, 0)


def _run_spmm(h0_flat, src2d, dst2d, w2d, zeros):
    mesh = plsc.VectorSubcoreMesh(core_axis_name="c", subcore_axis_name="s")
    kfn = pl.kernel(
        _spmm_body,
        out_type=(
            jax.ShapeDtypeStruct((FLAT, CHUNK), jnp.float32),
            jax.ShapeDtypeStruct((FLAT, CHUNK), jnp.float32),
        ),
        mesh=mesh,
        scratch_types=[
            pltpu.VMEM_SHARED((NNP, CHUNK), jnp.float32),
            pltpu.VMEM((BLK_E,), jnp.int32),
            pltpu.VMEM((BLK_E,), jnp.int32),
            pltpu.VMEM((BLK_E,), jnp.int32),
            pltpu.VMEM((BLK_E,), jnp.int32),
            pltpu.VMEM((BLK_E,), jnp.float32),
            pltpu.VMEM((BLK_E,), jnp.float32),
            pltpu.VMEM((SLAB_E, CHUNK), jnp.float32),
            pltpu.VMEM((SLAB_E, CHUNK), jnp.float32),
            pltpu.SemaphoreType.DMA,
            pltpu.SemaphoreType.DMA,
            pltpu.SemaphoreType.DMA,
            pltpu.SemaphoreType.DMA,
            pltpu.SemaphoreType.DMA,
            pltpu.SemaphoreType.DMA,
        ],
        compiler_params=pltpu.CompilerParams(use_tc_tiling_on_sc=False),
    )
    return kfn(h0_flat, src2d, dst2d, w2d, zeros)


def _fuse_body(h0_ref, h1_ref, h2_ref, mask_ref, fw_ref, fb_ref, ta_ref,
               out_ref):
    m = (h0_ref[...] + h1_ref[...] + h2_ref[...]) * (1.0 / 3.0)
    id_f = jnp.concatenate([m[0], m[1]], axis=1)
    text_f = jnp.concatenate([m[2], m[3]], axis=1)
    sig_a = jax.nn.sigmoid(ta_ref[0, 0])
    text_f = text_f * (1.0 + mask_ref[...] * sig_a)
    cat = jnp.concatenate([id_f, text_f], axis=1)
    gate = jax.nn.sigmoid(
        jnp.dot(cat, fw_ref[...], preferred_element_type=jnp.float32)
        + fb_ref[...])
    out_ref[...] = gate * id_f + (1.0 - gate) * text_f


def _fuse(h0, h1, h2, maskf, fw, fb, ta):
    r = 2000
    grid = N_NODES_K // r
    return pl.pallas_call(
        _fuse_body,
        grid=(grid,),
        in_specs=[
            pl.BlockSpec((NPLANES, r, CHUNK), lambda i: (0, i, 0)),
            pl.BlockSpec((NPLANES, r, CHUNK), lambda i: (0, i, 0)),
            pl.BlockSpec((NPLANES, r, CHUNK), lambda i: (0, i, 0)),
            pl.BlockSpec((r, 1), lambda i: (i, 0)),
            pl.BlockSpec((2 * EMB_K, EMB_K), lambda i: (0, 0)),
            pl.BlockSpec((1, EMB_K), lambda i: (0, 0)),
            pl.BlockSpec((1, 1), lambda i: (0, 0)),
        ],
        out_specs=pl.BlockSpec((r, EMB_K), lambda i: (i, 0)),
        out_shape=jax.ShapeDtypeStruct((N_NODES_K, EMB_K), jnp.float32),
    )(h0, h1, h2, maskf, fw, fb.reshape(1, EMB_K), ta.reshape(1, 1))


def kernel(edge_index, edge_weight, text_feats, tail_mask, user_emb, item_emb,
           text_proj_W, text_proj_b, fusion_W, fusion_b, tail_amp):
    emb_id = jnp.concatenate([user_emb, item_emb], axis=0)
    h0 = _build_planes(text_feats, emb_id, text_proj_W, text_proj_b)

    pad = E_PAD - E_K
    src1d = jnp.pad(edge_index[1], (0, pad))
    dst1d = jnp.pad(edge_index[0], (0, pad))
    w1d = jnp.pad(edge_weight, (0, pad))
    zeros = jnp.zeros((NNP, CHUNK), jnp.float32)

    h1f, h2f = _run_spmm(h0.reshape(FLAT, CHUNK), src1d, dst1d, w1d, zeros)
    h1 = h1f.reshape(NPLANES, NNP, CHUNK)
    h2 = h2f.reshape(NPLANES, NNP, CHUNK)

    maskf = tail_mask.astype(jnp.float32).reshape(N_NODES_K, 1)
    return _fuse(h0, h1, h2, maskf, fusion_W, fusion_b, tail_amp)


# speedup vs baseline: 1.1754x; 1.0348x over previous
import functools

import jax
import jax.numpy as jnp
from jax import lax
from jax.experimental import pallas as pl
from jax.experimental.pallas import tpu as pltpu
from jax.experimental.pallas import tpu_sc as plsc

N_USERS_K = 25000
N_NODES_K = 50000
E_K = 800000
EMB_K = 64
TEXT_K = 384
CHUNK = 32
NPLANES = 4

NC = 2
NS = 16
LANES = 16

NNP = 50048
FLAT = NPLANES * NNP
ROWS_PT = NNP // NS

MB = 128
E_PAD = 819200
EPT = E_PAD // NS
BLK_E = 1280
OUTER = EPT // BLK_E
SLAB_E = 320
SLABS = BLK_E // SLAB_E


def _planes_body(text_ref, id_ref, w_ref, b_ref, out_ref):
    t = jnp.dot(text_ref[...], w_ref[...], preferred_element_type=jnp.float32)
    t = t + b_ref[...]
    eid = id_ref[...]
    out_ref[0] = eid[:, :CHUNK]
    out_ref[1] = eid[:, CHUNK:]
    out_ref[2] = t[:, :CHUNK]
    out_ref[3] = t[:, CHUNK:]


def _build_planes(text_feats, emb_id, w, b):
    r = 2000
    grid = N_NODES_K // r
    return pl.pallas_call(
        _planes_body,
        grid=(grid,),
        in_specs=[
            pl.BlockSpec((r, TEXT_K), lambda i: (i, 0)),
            pl.BlockSpec((r, EMB_K), lambda i: (i, 0)),
            pl.BlockSpec((TEXT_K, EMB_K), lambda i: (0, 0)),
            pl.BlockSpec((1, EMB_K), lambda i: (0, 0)),
        ],
        out_specs=pl.BlockSpec((NPLANES, r, CHUNK), lambda i: (0, i, 0)),
        out_shape=jax.ShapeDtypeStruct((NPLANES, NNP, CHUNK), jnp.float32),
    )(text_feats, emb_id, w, b.reshape(1, EMB_K))


def _spmm_body(h0_ref, src_ref, dst_ref, w_ref, zeros_ref, h1_ref, h2_ref,
               acc, srcb0, srcb1, dstb0, dstb1, wb0, wb1, rows0, rows1,
               gsem0, gsem1, ssem0, ssem1, isem0, isem1):
    c = lax.axis_index("c")
    s = lax.axis_index("s")
    rows_bufs = (rows0, rows1)
    srcbs = (srcb0, srcb1)
    dstbs = (dstb0, dstb1)
    wbs = (wb0, wb1)
    gsems = (gsem0, gsem1)
    ssems = (ssem0, ssem1)
    isems = (isem0, isem1)

    def load_idx(o, p, rowbase, sync):
        e0 = s * EPT + o * BLK_E
        if sync:
            pltpu.sync_copy(src_ref.at[pl.ds(e0, BLK_E)], srcbs[p])
            pltpu.sync_copy(dst_ref.at[pl.ds(e0, BLK_E)], dstbs[p])
            pltpu.sync_copy(w_ref.at[pl.ds(e0, BLK_E)], wbs[p])
        else:
            pltpu.async_copy(src_ref.at[pl.ds(e0, BLK_E)], srcbs[p], isems[p])
            pltpu.async_copy(dst_ref.at[pl.ds(e0, BLK_E)], dstbs[p], isems[p])
            pltpu.async_copy(w_ref.at[pl.ds(e0, BLK_E)], wbs[p], isems[p])

    def wait_idx(o, p):
        e0 = s * EPT + o * BLK_E
        pltpu.make_async_copy(src_ref.at[pl.ds(e0, BLK_E)], srcbs[p],
                              isems[p]).wait()
        pltpu.make_async_copy(dst_ref.at[pl.ds(e0, BLK_E)], dstbs[p],
                              isems[p]).wait()
        pltpu.make_async_copy(w_ref.at[pl.ds(e0, BLK_E)], wbs[p],
                              isems[p]).wait()

    def add_base(p, rowbase):
        def body(i, _):
            sl = pl.ds(i * LANES, LANES)
            srcbs[p][sl] = srcbs[p][sl] + rowbase
            return 0
        lax.fori_loop(0, BLK_E // LANES, body, 0)

    def one_pass(in_ref, out_ref, rowbase):
        pltpu.sync_copy(zeros_ref.at[pl.ds(s * ROWS_PT, ROWS_PT)],
                        acc.at[pl.ds(s * ROWS_PT, ROWS_PT)])
        plsc.subcore_barrier()

        load_idx(0, 0, rowbase, sync=True)
        add_base(0, rowbase)
        pltpu.async_copy(in_ref.at[srcbs[0].at[pl.ds(0, SLAB_E)]],
                         rows_bufs[0], gsems[0])

        def outer_blk(o2, _):
            for p in range(2):
                o = o2 * 2 + p
                srcb = srcbs[p]
                dstb = dstbs[p]
                wb = wbs[p]

                @pl.when(o + 1 < OUTER)
                def _():
                    load_idx(o + 1, 1 - p, rowbase, sync=False)

                def micro2(j2, _, o=o, p=p, srcb=srcb, dstb=dstb, wb=wb):
                    for b in range(2):
                        j = j2 * 2 + b
                        rows = rows_bufs[b]
                        pltpu.make_async_copy(
                            in_ref.at[srcb.at[pl.ds(SLAB_E * j, SLAB_E)]],
                            rows, gsems[b]).wait()

                        @pl.when((o > 0) | (j > 0))
                        def _():
                            pltpu.make_async_copy(
                                rows_bufs[1 - b],
                                acc.at[dstb.at[pl.ds(0, SLAB_E)]],
                                ssems[1 - b]).wait()

                        @pl.when(j + 1 < SLABS)
                        def _():
                            pltpu.async_copy(
                                in_ref.at[srcb.at[pl.ds(SLAB_E * (j + 1),
                                                        SLAB_E)]],
                                rows_bufs[1 - b], gsems[1 - b])

                        @pl.when(j + 1 == SLABS)
                        def _():
                            @pl.when(o + 1 < OUTER)
                            def _():
                                wait_idx(o + 1, 1 - p)
                                add_base(1 - p, rowbase)
                                pltpu.async_copy(
                                    in_ref.at[srcbs[1 - p].at[
                                        pl.ds(0, SLAB_E)]],
                                    rows_bufs[1 - b], gsems[1 - b])

                        def mul_body(e16, _, j=j, wb=wb):
                            w16 = wb[pl.ds(SLAB_E * j + e16 * LANES, LANES)]
                            for u in range(LANES):
                                e = e16 * LANES + u
                                wv = w16[u]
                                lo = pl.ds(0, LANES)
                                hi = pl.ds(LANES, LANES)
                                rows[e, lo] = rows[e, lo] * wv
                                rows[e, hi] = rows[e, hi] * wv
                            return 0
                        lax.fori_loop(0, SLAB_E // LANES, mul_body, 0)

                        pltpu.async_copy(
                            rows, acc.at[dstb.at[pl.ds(SLAB_E * j, SLAB_E)]],
                            ssems[b], add=True)
                    return 0
                lax.fori_loop(0, SLABS // 2, micro2, 0)
            return 0
        lax.fori_loop(0, OUTER // 2, outer_blk, 0)
        pltpu.make_async_copy(
            rows_bufs[1], acc.at[dstbs[1].at[pl.ds(0, SLAB_E)]],
            ssems[1]).wait()
        plsc.subcore_barrier()

        pltpu.sync_copy(acc.at[pl.ds(s * ROWS_PT, ROWS_PT)],
                        out_ref.at[pl.ds(rowbase + s * ROWS_PT, ROWS_PT)])
        plsc.subcore_barrier()

    def chunk_body(k, _):
        rowbase = (2 * c + k) * NNP
        one_pass(h0_ref, h1_ref, rowbase)
        one_pass(h1_ref, h2_ref, rowbase)
        return 0
    lax.fori_loop(0, 2, chunk_body, 0)


def _run_spmm(h0_flat, src2d, dst2d, w2d, zeros):
    mesh = plsc.VectorSubcoreMesh(core_axis_name="c", subcore_axis_name="s")
    kfn = pl.kernel(
        _spmm_body,
        out_type=(
            jax.ShapeDtypeStruct((FLAT, CHUNK), jnp.float32),
            jax.ShapeDtypeStruct((FLAT, CHUNK), jnp.float32),
        ),
        mesh=mesh,
        scratch_types=[
            pltpu.VMEM_SHARED((NNP, CHUNK), jnp.float32),
            pltpu.VMEM((BLK_E,), jnp.int32),
            pltpu.VMEM((BLK_E,), jnp.int32),
            pltpu.VMEM((BLK_E,), jnp.int32),
            pltpu.VMEM((BLK_E,), jnp.int32),
            pltpu.VMEM((BLK_E,), jnp.float32),
            pltpu.VMEM((BLK_E,), jnp.float32),
            pltpu.VMEM((SLAB_E, CHUNK), jnp.float32),
            pltpu.VMEM((SLAB_E, CHUNK), jnp.float32),
            pltpu.SemaphoreType.DMA,
            pltpu.SemaphoreType.DMA,
            pltpu.SemaphoreType.DMA,
            pltpu.SemaphoreType.DMA,
            pltpu.SemaphoreType.DMA,
            pltpu.SemaphoreType.DMA,
        ],
        compiler_params=pltpu.CompilerParams(use_tc_tiling_on_sc=False),
    )
    return kfn(h0_flat, src2d, dst2d, w2d, zeros)


def _fuse_body(h0_ref, h1_ref, h2_ref, mask_ref, fw_ref, fb_ref, ta_ref,
               out_ref):
    m = (h0_ref[...] + h1_ref[...] + h2_ref[...]) * (1.0 / 3.0)
    id_f = jnp.concatenate([m[0], m[1]], axis=1)
    text_f = jnp.concatenate([m[2], m[3]], axis=1)
    sig_a = jax.nn.sigmoid(ta_ref[0, 0])
    text_f = text_f * (1.0 + mask_ref[...] * sig_a)
    cat = jnp.concatenate([id_f, text_f], axis=1)
    gate = jax.nn.sigmoid(
        jnp.dot(cat, fw_ref[...], preferred_element_type=jnp.float32)
        + fb_ref[...])
    out_ref[...] = gate * id_f + (1.0 - gate) * text_f


def _fuse(h0, h1, h2, maskf, fw, fb, ta):
    r = 2000
    grid = N_NODES_K // r
    return pl.pallas_call(
        _fuse_body,
        grid=(grid,),
        in_specs=[
            pl.BlockSpec((NPLANES, r, CHUNK), lambda i: (0, i, 0)),
            pl.BlockSpec((NPLANES, r, CHUNK), lambda i: (0, i, 0)),
            pl.BlockSpec((NPLANES, r, CHUNK), lambda i: (0, i, 0)),
            pl.BlockSpec((r, 1), lambda i: (i, 0)),
            pl.BlockSpec((2 * EMB_K, EMB_K), lambda i: (0, 0)),
            pl.BlockSpec((1, EMB_K), lambda i: (0, 0)),
            pl.BlockSpec((1, 1), lambda i: (0, 0)),
        ],
        out_specs=pl.BlockSpec((r, EMB_K), lambda i: (i, 0)),
        out_shape=jax.ShapeDtypeStruct((N_NODES_K, EMB_K), jnp.float32),
    )(h0, h1, h2, maskf, fw, fb.reshape(1, EMB_K), ta.reshape(1, 1))


def kernel(edge_index, edge_weight, text_feats, tail_mask, user_emb, item_emb,
           text_proj_W, text_proj_b, fusion_W, fusion_b, tail_amp):
    emb_id = jnp.concatenate([user_emb, item_emb], axis=0)
    h0 = _build_planes(text_feats, emb_id, text_proj_W, text_proj_b)

    pad = E_PAD - E_K
    src1d = jnp.pad(edge_index[1], (0, pad))
    dst1d = jnp.pad(edge_index[0], (0, pad))
    w1d = jnp.pad(edge_weight, (0, pad))
    zeros = jnp.zeros((NNP, CHUNK), jnp.float32)

    h1f, h2f = _run_spmm(h0.reshape(FLAT, CHUNK), src1d, dst1d, w1d, zeros)
    h1 = h1f.reshape(NPLANES, NNP, CHUNK)
    h2 = h2f.reshape(NPLANES, NNP, CHUNK)

    maskf = tail_mask.astype(jnp.float32).reshape(N_NODES_K, 1)
    return _fuse(h0, h1, h2, maskf, fusion_W, fusion_b, tail_amp)
